# baseline (speedup 1.0000x reference)
import functools

import jax
import jax.numpy as jnp
from jax import lax
from jax.experimental import pallas as pl
from jax.experimental.pallas import tpu as pltpu
from jax.experimental.pallas import tpu_sc as plsc

_NUM_QUERIES = 300
_HIDDEN = 256


@functools.cache
def _make_expand(batch: int):
    info = plsc.get_sparse_core_info()
    nc, ns = info.num_cores, info.num_subcores
    nw = nc * ns
    assert batch % nw == 0, (batch, nw)
    bpw = batch // nw

    mesh = plsc.VectorSubcoreMesh(core_axis_name="c", subcore_axis_name="s")

    @functools.partial(
        pl.kernel,
        mesh=mesh,
        out_type=jax.ShapeDtypeStruct(
            (batch, _NUM_QUERIES, _HIDDEN), jnp.float32
        ),
        scratch_types=[
            pltpu.VMEM_SHARED((bpw, _NUM_QUERIES, _HIDDEN), jnp.float32),
        ],
    )
    def k(table_hbm, out_hbm, shared):
        sid = lax.axis_index("s")
        cid = lax.axis_index("c")
        for i in range(bpw):

            @pl.when(sid == i)
            def _():
                pltpu.sync_copy(table_hbm, shared.at[i])

        plsc.subcore_barrier()
        wid = cid * ns + sid
        pltpu.sync_copy(shared, out_hbm.at[pl.ds(wid * bpw, bpw)])

    return k


def kernel(batch_ref, table):
    return _make_expand(batch_ref.shape[0])(table)
